# baseline (speedup 1.0000x reference)
import functools

import jax
import jax.numpy as jnp
from jax import lax
from jax.experimental import pallas as pl
from jax.experimental.pallas import tpu as pltpu
from jax.experimental.pallas import tpu_sc as plsc

N = 10000
E = 320000
D = 128

NC = 2
NS = 16
NW = NC * NS
L = 16

SB = 8
C = SB * 128
NPAD = 10240
STRIPE = NPAD // NS
EW = -(-E // (NW * C)) * C
EPAD = EW * NW
ITERS = EW // C
EBLKS = EW // 128


def _sc_scatter(row2d, col2d, attr2d, x):
    mesh = plsc.VectorSubcoreMesh(core_axis_name="c", subcore_axis_name="s")
    acc_t = jax.ShapeDtypeStruct((NC, NPAD, D), jnp.float32)
    cnt_t = jax.ShapeDtypeStruct((NC, NPAD, L), jnp.float32)

    @functools.partial(
        pl.kernel,
        out_type=(acc_t, cnt_t),
        mesh=mesh,
        compiler_params=pltpu.CompilerParams(use_tc_tiling_on_sc=False),
        scratch_types=[
            pltpu.VMEM((2, 128, D), jnp.float32),
            pltpu.VMEM((SB, 128), jnp.int32),
            pltpu.VMEM((SB, 128), jnp.int32),
            pltpu.VMEM((SB, 128), jnp.float32),
            pltpu.VMEM((128, L), jnp.float32),
            pltpu.VMEM_SHARED((NPAD, D), jnp.float32),
            pltpu.VMEM_SHARED((NPAD, L), jnp.float32),
            pltpu.SemaphoreType.DMA,
            pltpu.SemaphoreType.DMA,
            pltpu.SemaphoreType.DMA,
            pltpu.SemaphoreType.DMA,
            pltpu.SemaphoreType.DMA,
            pltpu.SemaphoreType.DMA,
        ],
    )
    def k(row_hbm, col_hbm, attr_hbm, x_hbm, acc_out, cnt_out,
          rows_v, rowi_v, coli_v, attr_v, ones_v, acc_sh, cnt_sh,
          sg0, sg1, ss0, ss1, sc0, sc1):
        cid = lax.axis_index("c")
        sid = lax.axis_index("s")
        wid = sid * NC + cid
        base = sid * STRIPE
        blk0 = wid * EBLKS
        sg = (sg0, sg1)
        ss = (ss0, ss1)
        sc = (sc0, sc1)

        @pl.loop(0, 128)
        def _(r):
            for kk in range(D // L):
                rows_v[0, r, pl.ds(kk * L, L)] = jnp.zeros((L,), jnp.float32)
            ones_v[r, :] = jnp.ones((L,), jnp.float32)

        for t in range(STRIPE // 128):
            pltpu.sync_copy(rows_v.at[0], acc_sh.at[pl.ds(base + t * 128, 128)])
            pltpu.sync_copy(ones_v, cnt_sh.at[pl.ds(base + t * 128, 128)])

        plsc.subcore_barrier()

        def scale(j, slot):
            @pl.loop(0, 128, step=L)
            def _(i16):
                av = attr_v[j, pl.ds(i16, L)]
                for t in range(L):
                    s = av[t]
                    r = i16 + t
                    for kk in range(D // L):
                        sl = pl.ds(kk * L, L)
                        rows_v[slot, r, sl] = rows_v[slot, r, sl] * s

        @pl.loop(0, ITERS)
        def _(it):
            b = blk0 + it * SB
            pltpu.sync_copy(row_hbm.at[pl.ds(b, SB)], rowi_v)
            pltpu.sync_copy(col_hbm.at[cid, pl.ds(b, SB)], coli_v)
            pltpu.sync_copy(attr_hbm.at[pl.ds(b, SB)], attr_v)

            hg = [None, None]
            hs = [None, None]
            hc = [None, None]
            prev = None
            for j in range(SB):
                slot = j % 2
                if hs[slot] is not None:
                    hs[slot].wait()
                    hc[slot].wait()
                hg[slot] = pltpu.async_copy(
                    x_hbm.at[coli_v.at[j]], rows_v.at[slot], sg[slot])
                if prev is not None:
                    pj, pslot = prev
                    hg[pslot].wait()
                    scale(pj, pslot)
                    hs[pslot] = pltpu.async_copy(
                        rows_v.at[pslot], acc_sh.at[rowi_v.at[pj]],
                        ss[pslot], add=True)
                    hc[pslot] = pltpu.async_copy(
                        ones_v, cnt_sh.at[rowi_v.at[pj]],
                        sc[pslot], add=True)
                prev = (j, slot)
            pj, pslot = prev
            hg[pslot].wait()
            scale(pj, pslot)
            hs[pslot] = pltpu.async_copy(
                rows_v.at[pslot], acc_sh.at[rowi_v.at[pj]],
                ss[pslot], add=True)
            hc[pslot] = pltpu.async_copy(
                ones_v, cnt_sh.at[rowi_v.at[pj]], sc[pslot], add=True)
            for slot in (0, 1):
                hs[slot].wait()
                hc[slot].wait()

        plsc.subcore_barrier()

        pltpu.sync_copy(acc_sh.at[pl.ds(base, STRIPE)],
                        acc_out.at[cid, pl.ds(base, STRIPE)])
        pltpu.sync_copy(cnt_sh.at[pl.ds(base, STRIPE)],
                        cnt_out.at[cid, pl.ds(base, STRIPE)])

    return k(row2d, col2d, attr2d, x)


def _tc_mlp(acc, cnt, x, emb, W_msg, W_upd):
    BN = 1000
    G = N // BN
    dn = (((1,), (1,)), ((), ()))
    hi = lax.Precision.HIGHEST

    def body(acc_r, cnt_r, x_r, emb_r, wm_r, wu_r, o_r):
        s = acc_r[0] + acc_r[1]
        c = cnt_r[0, :, 0:1] + cnt_r[1, :, 0:1] - 2.0
        xa = s / jnp.maximum(c, 1.0)
        wm = wm_r[...]
        wu = wu_r[...]
        m = jnp.maximum(
            lax.dot_general(xa, wm[:, :D], dn, precision=hi)
            + lax.dot_general(emb_r[...], wm[:, D:], dn, precision=hi), 0.0)
        o_r[...] = jnp.maximum(
            lax.dot_general(x_r[...], wu[:, :D], dn, precision=hi)
            + lax.dot_general(m, wu[:, D:], dn, precision=hi), 0.0)

    return pl.pallas_call(
        body,
        grid=(G,),
        in_specs=[
            pl.BlockSpec((NC, BN, D), lambda i: (0, i, 0)),
            pl.BlockSpec((NC, BN, L), lambda i: (0, i, 0)),
            pl.BlockSpec((BN, D), lambda i: (i, 0)),
            pl.BlockSpec((BN, D), lambda i: (i, 0)),
            pl.BlockSpec((D, 2 * D), lambda i: (0, 0)),
            pl.BlockSpec((D, 2 * D), lambda i: (0, 0)),
        ],
        out_specs=pl.BlockSpec((BN, D), lambda i: (i, 0)),
        out_shape=jax.ShapeDtypeStruct((N, D), jnp.float32),
    )(acc, cnt, x, emb, W_msg, W_upd)


def kernel(x, edge_index, edge_attr, x_agg_emb, W_msg, W_upd):
    row = edge_index[0].astype(jnp.int32)
    col = edge_index[1].astype(jnp.int32)
    attr = edge_attr[:, 0].astype(jnp.float32)
    pad = EPAD - E
    row_p = jnp.concatenate(
        [row, jnp.full((pad,), NPAD - 1, jnp.int32)]).reshape(EPAD // 128, 128)
    col_p = jnp.concatenate(
        [col, jnp.zeros((pad,), jnp.int32)]).reshape(EPAD // 128, 128)
    col_p = jnp.stack([col_p, col_p + N])
    xg = jnp.concatenate([x, x], axis=0)
    attr_p = jnp.concatenate(
        [attr, jnp.zeros((pad,), jnp.float32)]).reshape(EPAD // 128, 128)
    acc, cnt = _sc_scatter(row_p, col_p, attr_p, xg)
    return _tc_mlp(acc, cnt, x, x_agg_emb, W_msg, W_upd)


# speedup vs baseline: 1.3349x; 1.3349x over previous
import dataclasses
import functools

import numpy as np
import jax
import jax.numpy as jnp
from jax import lax
from jax.experimental import pallas as pl
from jax.experimental.pallas import tpu as pltpu
from jax.experimental.pallas import tpu_sc as plsc

N = 10000
E = 320000
D = 128

NC = 2
NS = 16
NW = NC * NS
L = 16

B2 = 64
SBK = 8
CH = SBK * B2
NPAD = 10240
STRIPE = NPAD // NS
EW = -(-E // (NW * CH)) * CH
EPAD = EW * NW
ITERS = EW // CH
BLKS = EW // B2

_PERM = np.empty((D,), dtype=np.int32)
for _m in range(D // 32):
    for _t in range(16):
        _PERM[32 * _m + 2 * _t] = 32 * _m + _t
        _PERM[32 * _m + 2 * _t + 1] = 32 * _m + 16 + _t


def _sc_compiler_params():
    cp = pltpu.CompilerParams(use_tc_tiling_on_sc=False)
    if "needs_layout_passes" in pltpu.CompilerParams.__dataclass_fields__:
        cp = dataclasses.replace(cp, needs_layout_passes=False)
    return cp


def _sc_scatter(row2d, col2d, attr2d, xbf):
    mesh = plsc.VectorSubcoreMesh(core_axis_name="c", subcore_axis_name="s")
    acc_t = jax.ShapeDtypeStruct((NC, NPAD, D), jnp.float32)
    cnt_t = jax.ShapeDtypeStruct((NC, NPAD, L), jnp.float32)

    @functools.partial(
        pl.kernel,
        out_type=(acc_t, cnt_t),
        mesh=mesh,
        compiler_params=_sc_compiler_params(),
        scratch_types=[
            pltpu.VMEM((2, B2, D), jnp.bfloat16),
            pltpu.VMEM((2, B2, D), jnp.float32),
            pltpu.VMEM((SBK, B2), jnp.int32),
            pltpu.VMEM((SBK, B2), jnp.int32),
            pltpu.VMEM((SBK, B2), jnp.float32),
            pltpu.VMEM((B2, L), jnp.float32),
            pltpu.VMEM_SHARED((NPAD, D), jnp.float32),
            pltpu.VMEM_SHARED((NPAD, L), jnp.float32),
            pltpu.SemaphoreType.DMA,
            pltpu.SemaphoreType.DMA,
            pltpu.SemaphoreType.DMA,
            pltpu.SemaphoreType.DMA,
            pltpu.SemaphoreType.DMA,
            pltpu.SemaphoreType.DMA,
        ],
    )
    def k(row_hbm, col_hbm, attr_hbm, x_hbm, acc_out, cnt_out,
          brows_v, frows_v, rowi_v, coli_v, attr_v, ones_v, acc_sh, cnt_sh,
          sg0, sg1, ss0, ss1, sc0, sc1):
        cid = lax.axis_index("c")
        sid = lax.axis_index("s")
        wid = sid * NC + cid
        base = sid * STRIPE
        blk0 = wid * BLKS
        sg = (sg0, sg1)
        ss = (ss0, ss1)
        sc = (sc0, sc1)

        @pl.loop(0, B2)
        def _(r):
            for kk in range(D // L):
                frows_v[0, r, pl.ds(kk * L, L)] = jnp.zeros((L,), jnp.float32)

        @pl.loop(0, B2)
        def _(r):
            ones_v[r, :] = jnp.ones((L,), jnp.float32)

        for t in range(STRIPE // B2):
            pltpu.sync_copy(frows_v.at[0],
                            acc_sh.at[pl.ds(base + t * B2, B2)])
        for t in range(STRIPE // B2):
            pltpu.sync_copy(ones_v, cnt_sh.at[pl.ds(base + t * B2, B2)])

        plsc.subcore_barrier()

        def scale(j, slot):
            @pl.loop(0, B2, step=L)
            def _(i16):
                av = attr_v[j, pl.ds(i16, L)]
                for t in range(L):
                    s = av[t]
                    r = i16 + t
                    for m in range(D // 32):
                        v = brows_v[slot, r, pl.ds(32 * m, 32)]
                        a, b = plsc.unpack(
                            v, format=plsc.PackFormat.INTERLEAVED)
                        frows_v[slot, r, pl.ds(32 * m, L)] = a * s
                        frows_v[slot, r, pl.ds(32 * m + L, L)] = b * s

        @pl.loop(0, ITERS)
        def _(it):
            b = blk0 + it * SBK
            pltpu.sync_copy(row_hbm.at[pl.ds(b, SBK)], rowi_v)
            pltpu.sync_copy(col_hbm.at[pl.ds(b, SBK)], coli_v)
            pltpu.sync_copy(attr_hbm.at[pl.ds(b, SBK)], attr_v)

            hg = [None, None]
            hs = [None, None]
            hc = [None, None]
            prev = None
            for j in range(SBK):
                slot = j % 2
                if hs[slot] is not None:
                    hs[slot].wait()
                    hc[slot].wait()
                hg[slot] = pltpu.async_copy(
                    x_hbm.at[coli_v.at[j]], brows_v.at[slot], sg[slot])
                if prev is not None:
                    pj, pslot = prev
                    hg[pslot].wait()
                    scale(pj, pslot)
                    hs[pslot] = pltpu.async_copy(
                        frows_v.at[pslot], acc_sh.at[rowi_v.at[pj]],
                        ss[pslot], add=True)
                    hc[pslot] = pltpu.async_copy(
                        ones_v, cnt_sh.at[rowi_v.at[pj]],
                        sc[pslot], add=True)
                prev = (j, slot)
            pj, pslot = prev
            hg[pslot].wait()
            scale(pj, pslot)
            hs[pslot] = pltpu.async_copy(
                frows_v.at[pslot], acc_sh.at[rowi_v.at[pj]],
                ss[pslot], add=True)
            hc[pslot] = pltpu.async_copy(
                ones_v, cnt_sh.at[rowi_v.at[pj]], sc[pslot], add=True)
            for slot in (0, 1):
                if hs[slot] is not None:
                    hs[slot].wait()
                    hc[slot].wait()

        plsc.subcore_barrier()

        pltpu.sync_copy(acc_sh.at[pl.ds(base, STRIPE)],
                        acc_out.at[cid, pl.ds(base, STRIPE)])
        pltpu.sync_copy(cnt_sh.at[pl.ds(base, STRIPE)],
                        cnt_out.at[cid, pl.ds(base, STRIPE)])

    return k(row2d, col2d, attr2d, xbf)


def _tc_mlp(acc, cnt, x, emb, W_msg, W_upd):
    BN = 1000
    G = N // BN
    dn = (((1,), (1,)), ((), ()))
    hi = lax.Precision.HIGHEST

    def body(acc_r, cnt_r, x_r, emb_r, wm_r, wu_r, o_r):
        s = acc_r[0] + acc_r[1]
        c = cnt_r[0, :, 0:1] + cnt_r[1, :, 0:1] - 2.0
        xa = s / jnp.maximum(c, 1.0)
        wm = wm_r[...]
        wu = wu_r[...]
        m = jnp.maximum(
            lax.dot_general(xa, wm[:, :D], dn, precision=hi)
            + lax.dot_general(emb_r[...], wm[:, D:], dn, precision=hi), 0.0)
        o_r[...] = jnp.maximum(
            lax.dot_general(x_r[...], wu[:, :D], dn, precision=hi)
            + lax.dot_general(m, wu[:, D:], dn, precision=hi), 0.0)

    return pl.pallas_call(
        body,
        grid=(G,),
        in_specs=[
            pl.BlockSpec((NC, BN, D), lambda i: (0, i, 0)),
            pl.BlockSpec((NC, BN, L), lambda i: (0, i, 0)),
            pl.BlockSpec((BN, D), lambda i: (i, 0)),
            pl.BlockSpec((BN, D), lambda i: (i, 0)),
            pl.BlockSpec((D, 2 * D), lambda i: (0, 0)),
            pl.BlockSpec((D, 2 * D), lambda i: (0, 0)),
        ],
        out_specs=pl.BlockSpec((BN, D), lambda i: (i, 0)),
        out_shape=jax.ShapeDtypeStruct((N, D), jnp.float32),
    )(acc, cnt, x, emb, W_msg, W_upd)


def kernel(x, edge_index, edge_attr, x_agg_emb, W_msg, W_upd):
    row = edge_index[0].astype(jnp.int32)
    col = edge_index[1].astype(jnp.int32)
    attr = edge_attr[:, 0].astype(jnp.float32)
    pad = EPAD - E
    row_p = jnp.concatenate(
        [row, jnp.full((pad,), NPAD - 1, jnp.int32)]).reshape(EPAD // B2, B2)
    col_p = jnp.concatenate(
        [col, jnp.zeros((pad,), jnp.int32)]).reshape(EPAD // B2, B2)
    attr_p = jnp.concatenate(
        [attr, jnp.zeros((pad,), jnp.float32)]).reshape(EPAD // B2, B2)
    xbf = x[:, jnp.asarray(_PERM)].astype(jnp.bfloat16)
    acc, cnt = _sc_scatter(row_p, col_p, attr_p, xbf)
    return _tc_mlp(acc, cnt, x, x_agg_emb, W_msg, W_upd)


# speedup vs baseline: 1.5475x; 1.1592x over previous
import dataclasses
import functools

import numpy as np
import jax
import jax.numpy as jnp
from jax import lax
from jax.experimental import pallas as pl
from jax.experimental.pallas import tpu as pltpu
from jax.experimental.pallas import tpu_sc as plsc

N = 10000
E = 320000
D = 128

NC = 2
NS = 16
NW = NC * NS
L = 16

B2 = 64
SBK = 16
CH = SBK * B2
NPAD = 10240
STRIPE = NPAD // NS
EW = -(-E // (NW * CH)) * CH
EPAD = EW * NW
ITERS = EW // CH
BLKS = EW // B2
GDEPTH = 4

_PERM = np.empty((D,), dtype=np.int32)
for _m in range(D // 32):
    for _t in range(16):
        _PERM[32 * _m + 2 * _t] = 32 * _m + _t
        _PERM[32 * _m + 2 * _t + 1] = 32 * _m + 16 + _t


def _sc_compiler_params():
    cp = pltpu.CompilerParams(use_tc_tiling_on_sc=False)
    if "needs_layout_passes" in pltpu.CompilerParams.__dataclass_fields__:
        cp = dataclasses.replace(cp, needs_layout_passes=False)
    return cp


def _sc_scatter(row2d, col2d, attr2d, xbf):
    mesh = plsc.VectorSubcoreMesh(core_axis_name="c", subcore_axis_name="s")
    acc_t = jax.ShapeDtypeStruct((NC, NPAD, D), jnp.float32)
    cnt_t = jax.ShapeDtypeStruct((NC, NPAD, L), jnp.float32)

    @functools.partial(
        pl.kernel,
        out_type=(acc_t, cnt_t),
        mesh=mesh,
        compiler_params=_sc_compiler_params(),
        scratch_types=[
            pltpu.VMEM((GDEPTH, B2, D), jnp.bfloat16),
            pltpu.VMEM((2, B2, D), jnp.float32),
            pltpu.VMEM((SBK, B2), jnp.int32),
            pltpu.VMEM((SBK, B2), jnp.int32),
            pltpu.VMEM((SBK, B2), jnp.float32),
            pltpu.VMEM((B2, L), jnp.float32),
            pltpu.VMEM_SHARED((NPAD, D), jnp.float32),
            pltpu.VMEM_SHARED((NPAD, L), jnp.float32),
            pltpu.SemaphoreType.DMA,
            pltpu.SemaphoreType.DMA,
            pltpu.SemaphoreType.DMA,
            pltpu.SemaphoreType.DMA,
            pltpu.SemaphoreType.DMA,
            pltpu.SemaphoreType.DMA,
            pltpu.SemaphoreType.DMA,
            pltpu.SemaphoreType.DMA,
        ],
    )
    def k(row_hbm, col_hbm, attr_hbm, x_hbm, acc_out, cnt_out,
          brows_v, frows_v, rowi_v, coli_v, attr_v, ones_v, acc_sh, cnt_sh,
          sg0, sg1, sg2, sg3, ss0, ss1, sc0, sc1):
        cid = lax.axis_index("c")
        sid = lax.axis_index("s")
        wid = sid * NC + cid
        base = sid * STRIPE
        blk0 = wid * BLKS
        sg = (sg0, sg1, sg2, sg3)
        ss = (ss0, ss1)
        sc = (sc0, sc1)

        @pl.loop(0, B2)
        def _(r):
            for kk in range(D // L):
                frows_v[0, r, pl.ds(kk * L, L)] = jnp.zeros((L,), jnp.float32)

        @pl.loop(0, B2)
        def _(r):
            ones_v[r, :] = jnp.ones((L,), jnp.float32)

        for t in range(STRIPE // B2):
            pltpu.sync_copy(frows_v.at[0],
                            acc_sh.at[pl.ds(base + t * B2, B2)])
        for t in range(STRIPE // B2):
            pltpu.sync_copy(ones_v, cnt_sh.at[pl.ds(base + t * B2, B2)])

        plsc.subcore_barrier()

        def scale(j, gslot, fslot):
            @pl.loop(0, B2, step=L)
            def _(i16):
                av = attr_v[j, pl.ds(i16, L)]
                for t in range(L):
                    s = av[t]
                    r = i16 + t
                    for m in range(D // 32):
                        v = brows_v[gslot, r, pl.ds(32 * m, 32)]
                        a, b = plsc.unpack(
                            v, format=plsc.PackFormat.INTERLEAVED)
                        frows_v[fslot, r, pl.ds(32 * m, L)] = a * s
                        frows_v[fslot, r, pl.ds(32 * m + L, L)] = b * s

        @pl.loop(0, ITERS)
        def _(it):
            b = blk0 + it * SBK
            pltpu.sync_copy(row_hbm.at[pl.ds(b, SBK)], rowi_v)
            pltpu.sync_copy(col_hbm.at[pl.ds(b, SBK)], coli_v)
            pltpu.sync_copy(attr_hbm.at[pl.ds(b, SBK)], attr_v)

            hg = [None] * GDEPTH
            hs = [None, None]
            hc = [None, None]
            for q in range(GDEPTH - 1):
                hg[q] = pltpu.async_copy(
                    x_hbm.at[coli_v.at[q]], brows_v.at[q], sg[q])
            for j in range(SBK):
                gslot = j % GDEPTH
                fslot = j % 2
                nxt = j + GDEPTH - 1
                if nxt < SBK:
                    ns = nxt % GDEPTH
                    hg[ns] = pltpu.async_copy(
                        x_hbm.at[coli_v.at[nxt]], brows_v.at[ns], sg[ns])
                hg[gslot].wait()
                if hs[fslot] is not None:
                    hs[fslot].wait()
                    hc[fslot].wait()
                scale(j, gslot, fslot)
                hs[fslot] = pltpu.async_copy(
                    frows_v.at[fslot], acc_sh.at[rowi_v.at[j]],
                    ss[fslot], add=True)
                hc[fslot] = pltpu.async_copy(
                    ones_v, cnt_sh.at[rowi_v.at[j]],
                    sc[fslot], add=True)
            for fslot in (0, 1):
                hs[fslot].wait()
                hc[fslot].wait()

        plsc.subcore_barrier()

        pltpu.sync_copy(acc_sh.at[pl.ds(base, STRIPE)],
                        acc_out.at[cid, pl.ds(base, STRIPE)])
        pltpu.sync_copy(cnt_sh.at[pl.ds(base, STRIPE)],
                        cnt_out.at[cid, pl.ds(base, STRIPE)])

    return k(row2d, col2d, attr2d, xbf)


def _tc_mlp(acc, cnt, x, emb, W_msg, W_upd):
    BN = 1000
    G = N // BN
    dn = (((1,), (1,)), ((), ()))
    hi = lax.Precision.HIGHEST

    def body(acc_r, cnt_r, x_r, emb_r, wm_r, wu_r, o_r):
        s = acc_r[0] + acc_r[1]
        c = cnt_r[0, :, 0:1] + cnt_r[1, :, 0:1] - 2.0
        xa = s / jnp.maximum(c, 1.0)
        wm = wm_r[...]
        wu = wu_r[...]
        m = jnp.maximum(
            lax.dot_general(xa, wm[:, :D], dn, precision=hi)
            + lax.dot_general(emb_r[...], wm[:, D:], dn, precision=hi), 0.0)
        o_r[...] = jnp.maximum(
            lax.dot_general(x_r[...], wu[:, :D], dn, precision=hi)
            + lax.dot_general(m, wu[:, D:], dn, precision=hi), 0.0)

    return pl.pallas_call(
        body,
        grid=(G,),
        in_specs=[
            pl.BlockSpec((NC, BN, D), lambda i: (0, i, 0)),
            pl.BlockSpec((NC, BN, L), lambda i: (0, i, 0)),
            pl.BlockSpec((BN, D), lambda i: (i, 0)),
            pl.BlockSpec((BN, D), lambda i: (i, 0)),
            pl.BlockSpec((D, 2 * D), lambda i: (0, 0)),
            pl.BlockSpec((D, 2 * D), lambda i: (0, 0)),
        ],
        out_specs=pl.BlockSpec((BN, D), lambda i: (i, 0)),
        out_shape=jax.ShapeDtypeStruct((N, D), jnp.float32),
    )(acc, cnt, x, emb, W_msg, W_upd)


def kernel(x, edge_index, edge_attr, x_agg_emb, W_msg, W_upd):
    row = edge_index[0].astype(jnp.int32)
    col = edge_index[1].astype(jnp.int32)
    attr = edge_attr[:, 0].astype(jnp.float32)
    pad = EPAD - E
    row_p = jnp.concatenate(
        [row, jnp.full((pad,), NPAD - 1, jnp.int32)]).reshape(EPAD // B2, B2)
    col_p = jnp.concatenate(
        [col, jnp.zeros((pad,), jnp.int32)]).reshape(EPAD // B2, B2)
    attr_p = jnp.concatenate(
        [attr, jnp.zeros((pad,), jnp.float32)]).reshape(EPAD // B2, B2)
    xbf = x[:, jnp.asarray(_PERM)].astype(jnp.bfloat16)
    acc, cnt = _sc_scatter(row_p, col_p, attr_p, xbf)
    return _tc_mlp(acc, cnt, x, x_agg_emb, W_msg, W_upd)
